# Initial kernel scaffold; baseline (speedup 1.0000x reference)
import jax
import jax.numpy as jnp
from jax.experimental import pallas as pl


def kernel(x, edge_index, batch, W1, b1, W2, b2, W3, b3, Wc, bc):
    raise NotImplementedError("write your pallas kernel here")


# speedup vs baseline: 33.6952x; 33.6952x over previous
import functools

import jax
import jax.numpy as jnp
from jax import lax
from jax.experimental import pallas as pl
from jax.experimental.pallas import tpu as pltpu
from jax.experimental.pallas import tpu_sc as plsc

NC = 2
NS = 16
NW = NC * NS


def _vmesh():
  return plsc.VectorSubcoreMesh(
      core_axis_name="c", subcore_axis_name="s", num_cores=NC, num_subcores=NS)


def _make_prep(n, e):
  ept = e // NW

  @functools.partial(
      pl.kernel,
      out_type=(
          jax.ShapeDtypeStruct((NW, n), jnp.float32),
          jax.ShapeDtypeStruct((e,), jnp.int32),
      ),
      mesh=_vmesh(),
      compiler_params=pltpu.CompilerParams(needs_layout_passes=False),
      scratch_types=[
          pltpu.VMEM((ept,), jnp.int32),
          pltpu.VMEM((ept,), jnp.int32),
          pltpu.VMEM((ept,), jnp.int32),
          pltpu.VMEM((n,), jnp.float32),
          pltpu.SemaphoreType.DMA,
      ],
  )
  def prep(src_hbm, dst_hbm, deg_out, pk_out, src_v, dst_v, pk_v, deg_v, sem):
    wid = lax.axis_index("s") * NC + lax.axis_index("c")
    base = wid * ept
    pltpu.make_async_copy(src_hbm.at[pl.ds(base, ept)], src_v, sem).start()
    pltpu.make_async_copy(dst_hbm.at[pl.ds(base, ept)], dst_v, sem).start()

    zeros = jnp.zeros((16,), jnp.float32)

    @plsc.parallel_loop(0, n, 16, unroll=5)
    def _(i):
      deg_v[pl.ds(i, 16)] = zeros

    pltpu.make_async_copy(src_hbm.at[pl.ds(base, ept)], src_v, sem).wait()
    pltpu.make_async_copy(dst_hbm.at[pl.ds(base, ept)], dst_v, sem).wait()

    ones = jnp.ones((16,), jnp.float32)

    @plsc.parallel_loop(0, ept, 16, unroll=5)
    def _(i):
      sv = src_v[pl.ds(i, 16)]
      dv = dst_v[pl.ds(i, 16)]
      pk_v[pl.ds(i, 16)] = jnp.bitwise_or(sv, lax.shift_left(dv, 16))
      plsc.addupdate_scatter(deg_v, [dv], ones)

    pltpu.sync_copy(deg_v, deg_out.at[wid])
    pltpu.sync_copy(pk_v, pk_out.at[pl.ds(base, ept)])

  return prep


def _make_agg(n, e, h, ch=3200, unroll=8):
  nch = e // ch
  assert nch * ch == e and nch % 2 == 0 and ch % 16 == 0

  @functools.partial(
      pl.kernel,
      out_type=jax.ShapeDtypeStruct((h, n), jnp.float32),
      mesh=_vmesh(),
      compiler_params=pltpu.CompilerParams(needs_layout_passes=False),
      scratch_types=[
          pltpu.VMEM((n,), jnp.float32),
          pltpu.VMEM((n,), jnp.float32),
          pltpu.VMEM((n,), jnp.float32),
          pltpu.VMEM((n,), jnp.float32),
          pltpu.VMEM((ch,), jnp.int32),
          pltpu.VMEM((ch,), jnp.int32),
          pltpu.SemaphoreType.DMA,
          pltpu.SemaphoreType.DMA,
          pltpu.SemaphoreType.DMA,
      ],
  )
  def agg(g_hbm, pk_hbm, acc_out, col0, col1, acc0, acc1, pkb0, pkb1,
          sem0, sem1, semc):
    wid = lax.axis_index("s") * NC + lax.axis_index("c")
    r0 = wid * 2
    pltpu.make_async_copy(g_hbm.at[r0], col0, semc).start()
    pltpu.make_async_copy(g_hbm.at[r0 + 1], col1, semc).start()
    pltpu.make_async_copy(pk_hbm.at[pl.ds(0, ch)], pkb0, sem0).start()
    pltpu.make_async_copy(pk_hbm.at[pl.ds(ch, ch)], pkb1, sem1).start()

    zeros = jnp.zeros((16,), jnp.float32)

    @plsc.parallel_loop(0, n, 16, unroll=5)
    def _(i):
      acc0[pl.ds(i, 16)] = zeros
      acc1[pl.ds(i, 16)] = zeros

    pltpu.make_async_copy(g_hbm.at[r0], col0, semc).wait()
    pltpu.make_async_copy(g_hbm.at[r0 + 1], col1, semc).wait()

    mask16 = jnp.int32(0xFFFF)

    def process(buf):
      @plsc.parallel_loop(0, ch, 16, unroll=unroll)
      def _(i):
        pk = buf[pl.ds(i, 16)]
        sv = jnp.bitwise_and(pk, mask16)
        dv = lax.shift_right_logical(pk, 16)
        plsc.addupdate_scatter(acc0, [dv], plsc.load_gather(col0, [sv]))
        plsc.addupdate_scatter(acc1, [dv], plsc.load_gather(col1, [sv]))

    def chunk_pair(k, carry):
      pltpu.make_async_copy(pk_hbm.at[pl.ds(0, ch)], pkb0, sem0).wait()
      process(pkb0)

      @pl.when(k < nch // 2 - 1)
      def _():
        pltpu.make_async_copy(
            pk_hbm.at[pl.ds((2 * k + 2) * ch, ch)], pkb0, sem0).start()

      pltpu.make_async_copy(pk_hbm.at[pl.ds(0, ch)], pkb1, sem1).wait()
      process(pkb1)

      @pl.when(k < nch // 2 - 1)
      def _():
        pltpu.make_async_copy(
            pk_hbm.at[pl.ds((2 * k + 3) * ch, ch)], pkb1, sem1).start()

      return carry

    lax.fori_loop(0, nch // 2, chunk_pair, 0)

    pltpu.sync_copy(acc0, acc_out.at[r0])
    pltpu.sync_copy(acc1, acc_out.at[r0 + 1])

  return agg


def _tc_first(deg_ref, x_ref, w_ref, dis_ref, g_ref):
  deg = jnp.sum(deg_ref[...], axis=0, keepdims=True) + 1.0
  dis = lax.rsqrt(deg)
  dis_ref[...] = dis
  hw = lax.dot_general(w_ref[...], x_ref[...], (((0,), (1,)), ((), ())),
                       preferred_element_type=jnp.float32)
  g_ref[...] = hw * dis


def _tc_mid(acc_ref, g_ref, dis_ref, b_ref, w_ref, gn_ref):
  dis = dis_ref[...]
  hidden = jnp.maximum(dis * (acc_ref[...] + g_ref[...]) + b_ref[...], 0.0)
  hw = lax.dot_general(w_ref[...], hidden, (((0,), (0,)), ((), ())),
                       preferred_element_type=jnp.float32)
  gn_ref[...] = hw * dis


def _make_tc_final(g_seg):
  def tc_final(acc_ref, g_ref, dis_ref, b_ref, batch_ref, wc_ref, bc_ref,
               out_ref):
    dis = dis_ref[...]
    hidden = jnp.maximum(dis * (acc_ref[...] + g_ref[...]) + b_ref[...], 0.0)
    n = hidden.shape[1]
    gid = lax.broadcasted_iota(jnp.int32, (g_seg, n), 0).astype(jnp.float32)
    member = (gid == batch_ref[...]).astype(jnp.float32)
    sums = lax.dot_general(hidden, member, (((1,), (1,)), ((), ())),
                           preferred_element_type=jnp.float32)
    cnts = jnp.sum(member, axis=1, keepdims=True)
    num = lax.dot_general(sums, wc_ref[...], (((0,), (0,)), ((), ())),
                          preferred_element_type=jnp.float32)
    out_ref[...] = num / jnp.maximum(cnts, 1.0) + bc_ref[...]

  return tc_final


def kernel(x, edge_index, batch, W1, b1, W2, b2, W3, b3, Wc, bc):
  n, d = x.shape
  h = W1.shape[1]
  c = Wc.shape[1]
  e = edge_index.shape[1]
  g_seg = 64

  src = edge_index[0].astype(jnp.int32)
  dst = edge_index[1].astype(jnp.int32)
  batch_f = batch.astype(jnp.float32).reshape(1, n)
  b1r = b1.reshape(h, 1)
  b2r = b2.reshape(h, 1)
  b3r = b3.reshape(h, 1)
  bcr = bc.reshape(1, c)

  deg_parts, packed = _make_prep(n, e)(src, dst)

  dis, g1 = pl.pallas_call(
      _tc_first,
      out_shape=(jax.ShapeDtypeStruct((1, n), jnp.float32),
                 jax.ShapeDtypeStruct((h, n), jnp.float32)),
  )(deg_parts, x, W1)

  agg = _make_agg(n, e, h)

  acc1 = agg(g1, packed)
  g2 = pl.pallas_call(
      _tc_mid, out_shape=jax.ShapeDtypeStruct((h, n), jnp.float32),
  )(acc1, g1, dis, b1r, W2)

  acc2 = agg(g2, packed)
  g3 = pl.pallas_call(
      _tc_mid, out_shape=jax.ShapeDtypeStruct((h, n), jnp.float32),
  )(acc2, g2, dis, b2r, W3)

  acc3 = agg(g3, packed)
  out = pl.pallas_call(
      _make_tc_final(g_seg),
      out_shape=jax.ShapeDtypeStruct((g_seg, c), jnp.float32),
  )(acc3, g3, dis, b3r, batch_f, Wc, bcr)

  return out
